# baseline (speedup 1.0000x reference)
import jax
import jax.numpy as jnp
from jax import lax
from jax.experimental import pallas as pl
from jax.experimental.pallas import tpu as pltpu
from jax.experimental.pallas import tpu_sc as plsc

N = 16_777_216
N_BINS = 20
NC = 2
NS = 16
NW = NC * NS
NP = N // NW
CH = 16_384
NCH = NP // CH
VPC = CH // 16
NBUF = 2


UNROLL = 16


def _hist_body(p_hbm, t_hbm, out_hbm, pbuf, tbuf, histv, histd, histc,
               sp0, sp1, st0, st1):
    cid = lax.axis_index("c")
    sid = lax.axis_index("s")
    wid = sid * NC + cid

    zero = jnp.zeros((16,), jnp.float32)
    for r in range(N_BINS):
        histv[pl.ds(16 * r, 16)] = zero
        histd[pl.ds(16 * r, 16)] = zero
        histc[pl.ds(16 * r, 16)] = zero

    sems_p = (sp0, sp1)
    sems_t = (st0, st1)

    def issue(slot, g):
        off = (g * NW + wid) * CH
        pltpu.async_copy(p_hbm.at[pl.ds(off, CH)], pbuf.at[slot], sems_p[slot])
        pltpu.async_copy(t_hbm.at[pl.ds(off, CH)], tbuf.at[slot], sems_t[slot])

    def wait(slot):
        pltpu.make_async_copy(p_hbm.at[pl.ds(0, CH)], pbuf.at[slot],
                              sems_p[slot]).wait()
        pltpu.make_async_copy(t_hbm.at[pl.ds(0, CH)], tbuf.at[slot],
                              sems_t[slot]).wait()

    scale = jnp.float32(N_BINS * (1.0 - 2.0 ** -22))
    lanes = lax.iota(jnp.int32, 16)
    one = jnp.float32(1.0)
    pack = jnp.float32(2.0 ** -12)
    unpack = jnp.float32(2.0 ** 12)
    half = jnp.float32(0.5)

    def consume(slot):
        def vbody(j, c):
            off0 = j * (16 * UNROLL)
            ps, ts = [], []
            for u in range(UNROLL):
                off = off0 + 16 * u
                ps.append(pbuf[slot, pl.ds(off, 16)])
                ts.append(tbuf[slot, pl.ds(off, 16)])
            idxs, vals = [], []
            for u in range(UNROLL):
                b = (ps[u] * scale).astype(jnp.int32)
                idxs.append(lax.shift_left(b, 4) + lanes)
                vals.append((ps[u] - ts[u]) * pack + one)
            for u in range(UNROLL):
                plsc.addupdate_scatter(histv, [idxs[u]], vals[u])
            return c

        lax.fori_loop(0, VPC // UNROLL, vbody, 0)
        for r in range(N_BINS):
            sl = pl.ds(16 * r, 16)
            s = histv[sl]
            cf = (s + half).astype(jnp.int32).astype(jnp.float32)
            plsc.addupdate(histd.at[sl], (s - cf) * unpack)
            plsc.addupdate(histc.at[sl], cf)
            histv[sl] = zero

    for s in range(NBUF):
        issue(s, s)

    def ring(it, c):
        for s in range(NBUF):
            wait(s)
            consume(s)
            issue(s, it * NBUF + s + NBUF)
        return c

    lax.fori_loop(0, NCH // NBUF - 1, ring, 0)
    for s in range(NBUF):
        wait(s)
        consume(s)

    pltpu.sync_copy(histd, out_hbm.at[wid, 0])
    pltpu.sync_copy(histc, out_hbm.at[wid, 1])


_hist = pl.kernel(
    _hist_body,
    out_type=jax.ShapeDtypeStruct((NW, 2, N_BINS * 16), jnp.float32),
    mesh=plsc.VectorSubcoreMesh(core_axis_name="c", subcore_axis_name="s"),
    compiler_params=pltpu.CompilerParams(needs_layout_passes=False),
    scratch_types=[
        pltpu.VMEM((NBUF, CH), jnp.float32),
        pltpu.VMEM((NBUF, CH), jnp.float32),
        pltpu.VMEM((N_BINS * 16,), jnp.float32),
        pltpu.VMEM((N_BINS * 16,), jnp.float32),
        pltpu.VMEM((N_BINS * 16,), jnp.float32),
    ] + [pltpu.SemaphoreType.DMA] * (2 * NBUF),
)


def _finalize_body(x_ref, o_ref):
    x = x_ref[...]
    a = jnp.sum(x, axis=0)
    sd = jnp.sum(a[0], axis=1, keepdims=True)
    cnt = jnp.sum(a[1], axis=1, keepdims=True)
    term = jnp.where(cnt > 0, jnp.abs(sd) / jnp.maximum(cnt, 1.0), 0.0)
    o_ref[0, 0] = jnp.sum(term) / jnp.float32(N_BINS)


_finalize = pl.pallas_call(
    _finalize_body,
    out_shape=jax.ShapeDtypeStruct((1, 1), jnp.float32),
    out_specs=pl.BlockSpec(memory_space=pltpu.SMEM),
)


def kernel(preds, targets):
    parts = _hist(preds.reshape(-1), targets.reshape(-1).astype(jnp.float32))
    return _finalize(parts.reshape(NW, 2, N_BINS, 16))[0, 0]


# speedup vs baseline: 1.0479x; 1.0479x over previous
import jax
import jax.numpy as jnp
from jax import lax
from jax.experimental import pallas as pl
from jax.experimental.pallas import tpu as pltpu
from jax.experimental.pallas import tpu_sc as plsc

N = 16_777_216
N_BINS = 20
NC = 2
NS = 16
NW = NC * NS
NP = N // NW
CH = 16_384
NCH = NP // CH
VPC = CH // 16
NBUF = 2


UNROLL = 16


def _hist_body(p_hbm, t_hbm, out_hbm, pbuf, tbuf, histv, histd, histc,
               sp0, sp1, st0, st1):
    cid = lax.axis_index("c")
    sid = lax.axis_index("s")
    wid = sid * NC + cid

    zero = jnp.zeros((16,), jnp.float32)
    for r in range(N_BINS):
        histv[pl.ds(16 * r, 16)] = zero
        histd[pl.ds(16 * r, 16)] = zero
        histc[pl.ds(16 * r, 16)] = zero

    sems_p = (sp0, sp1)
    sems_t = (st0, st1)

    def issue(slot, g):
        off = (g * NW + wid) * CH
        pltpu.async_copy(p_hbm.at[pl.ds(off, CH)], pbuf.at[slot], sems_p[slot])
        pltpu.async_copy(t_hbm.at[pl.ds(off, CH)], tbuf.at[slot], sems_t[slot])

    def wait(slot):
        pltpu.make_async_copy(p_hbm.at[pl.ds(0, CH)], pbuf.at[slot],
                              sems_p[slot]).wait()
        pltpu.make_async_copy(t_hbm.at[pl.ds(0, CH)], tbuf.at[slot],
                              sems_t[slot]).wait()

    scale = jnp.float32(N_BINS * (1.0 - 2.0 ** -22))
    lanes = lax.iota(jnp.int32, 16)
    one = jnp.float32(1.0)
    pack = jnp.float32(2.0 ** -12)
    unpack = jnp.float32(2.0 ** 12)
    half = jnp.float32(0.5)

    def consume(slot):
        def vbody(j, c):
            off0 = j * (16 * UNROLL)
            ps, ts = [], []
            for u in range(UNROLL):
                off = off0 + 16 * u
                ps.append(pbuf[slot, pl.ds(off, 16)])
                ts.append(tbuf[slot, pl.ds(off, 16)])
            idxs, vals = [], []
            for u in range(UNROLL):
                b = (ps[u] * scale).astype(jnp.int32)
                idxs.append(lax.shift_left(b, 4) + lanes)
                vals.append((ps[u] - ts[u]) * pack + one)
            for u in range(UNROLL):
                plsc.addupdate_scatter(histv, [idxs[u]], vals[u])
            return c

        lax.fori_loop(0, VPC // UNROLL, vbody, 0)
        ss = [histv[pl.ds(16 * r, 16)] for r in range(N_BINS)]
        cfs = [(s + half).astype(jnp.int32).astype(jnp.float32) for s in ss]
        sds = [(s - cf) * unpack for s, cf in zip(ss, cfs)]
        for r in range(N_BINS):
            sl = pl.ds(16 * r, 16)
            plsc.addupdate(histd.at[sl], sds[r])
            plsc.addupdate(histc.at[sl], cfs[r])
            histv[sl] = zero

    for s in range(NBUF):
        issue(s, s)

    def ring(it, c):
        for s in range(NBUF):
            wait(s)
            consume(s)
            issue(s, it * NBUF + s + NBUF)
        return c

    lax.fori_loop(0, NCH // NBUF - 1, ring, 0)
    for s in range(NBUF):
        wait(s)
        consume(s)

    pltpu.sync_copy(histd, out_hbm.at[wid, 0])
    pltpu.sync_copy(histc, out_hbm.at[wid, 1])


_hist = pl.kernel(
    _hist_body,
    out_type=jax.ShapeDtypeStruct((NW, 2, N_BINS * 16), jnp.float32),
    mesh=plsc.VectorSubcoreMesh(core_axis_name="c", subcore_axis_name="s"),
    compiler_params=pltpu.CompilerParams(needs_layout_passes=False),
    scratch_types=[
        pltpu.VMEM((NBUF, CH), jnp.float32),
        pltpu.VMEM((NBUF, CH), jnp.float32),
        pltpu.VMEM((N_BINS * 16,), jnp.float32),
        pltpu.VMEM((N_BINS * 16,), jnp.float32),
        pltpu.VMEM((N_BINS * 16,), jnp.float32),
    ] + [pltpu.SemaphoreType.DMA] * (2 * NBUF),
)


def _finalize_body(x_ref, o_ref):
    x = x_ref[...]
    a = jnp.sum(x, axis=0)
    sd = jnp.sum(a[0], axis=1, keepdims=True)
    cnt = jnp.sum(a[1], axis=1, keepdims=True)
    term = jnp.where(cnt > 0, jnp.abs(sd) / jnp.maximum(cnt, 1.0), 0.0)
    o_ref[0, 0] = jnp.sum(term) / jnp.float32(N_BINS)


_finalize = pl.pallas_call(
    _finalize_body,
    out_shape=jax.ShapeDtypeStruct((1, 1), jnp.float32),
    out_specs=pl.BlockSpec(memory_space=pltpu.SMEM),
)


def kernel(preds, targets):
    parts = _hist(preds.reshape(-1), targets.reshape(-1).astype(jnp.float32))
    return _finalize(parts.reshape(NW, 2, N_BINS, 16))[0, 0]
